# Initial kernel scaffold; baseline (speedup 1.0000x reference)
import jax
import jax.numpy as jnp
from jax.experimental import pallas as pl


def kernel(speaker_id, phone, train, speaker_id_table, phone_table):
    raise NotImplementedError("write your pallas kernel here")


# speedup vs baseline: 4.5048x; 4.5048x over previous
import functools

import jax
import jax.numpy as jnp
from jax import lax
from jax.experimental import pallas as pl
from jax.experimental.pallas import tpu as pltpu
from jax.experimental.pallas import tpu_sc as plsc

HIDDEN = 64
NC, NS = 2, 16
NW = NC * NS
CHUNK = 128
K = 8


@functools.partial(jax.jit, static_argnums=())
def _embed_pair(sid, ph, stab, ptab):
    nw, nch, chunk = sid.shape
    n = nw * nch * chunk
    per_w = nch * chunk
    nb = nch // K

    mesh = plsc.VectorSubcoreMesh(core_axis_name="c", subcore_axis_name="s")

    @functools.partial(
        pl.kernel,
        mesh=mesh,
        out_type=[
            jax.ShapeDtypeStruct((n, HIDDEN), jnp.float32),
            jax.ShapeDtypeStruct((n, HIDDEN), jnp.float32),
        ],
        scratch_types=[
            pltpu.VMEM((nch, chunk), jnp.int32),
            pltpu.VMEM((K, CHUNK, HIDDEN), jnp.float32),
            pltpu.SemaphoreType.DMA,
            pltpu.SemaphoreType.DMA,
        ],
        compiler_params=pltpu.CompilerParams(use_tc_tiling_on_sc=False),
    )
    def emb(sid_hbm, ph_hbm, stab_hbm, ptab_hbm, out_s, out_p,
            idx_v, rows_v, gsem, osem):
        wid = lax.axis_index("s") * NC + lax.axis_index("c")
        base = wid * per_w

        def run_table(idx_hbm, tab_hbm, out_hbm):
            pltpu.sync_copy(idx_hbm.at[wid], idx_v)

            def batch(bi, carry):
                c0 = bi * K
                for b in range(K):
                    pltpu.async_copy(
                        tab_hbm.at[idx_v.at[c0 + b]], rows_v.at[b], gsem)
                for b in range(K):
                    pltpu.make_async_copy(
                        tab_hbm.at[idx_v.at[c0 + b]], rows_v.at[b], gsem
                    ).wait()
                    pltpu.async_copy(
                        rows_v.at[b],
                        out_hbm.at[pl.ds(base + (c0 + b) * CHUNK, CHUNK)],
                        osem)
                for b in range(K):
                    pltpu.make_async_copy(
                        rows_v.at[b],
                        out_hbm.at[pl.ds(base + (c0 + b) * CHUNK, CHUNK)],
                        osem).wait()
                return carry

            lax.fori_loop(0, nb, batch, 0)

        run_table(sid_hbm, stab_hbm, out_s)
        run_table(ph_hbm, ptab_hbm, out_p)

    return emb(sid, ph, stab, ptab)


def kernel(speaker_id, phone, train, speaker_id_table, phone_table):
    del train
    b, l = speaker_id.shape
    n = b * l
    nch = n // (NW * CHUNK)
    sid = speaker_id.reshape(NW, nch, CHUNK)
    ph = phone.reshape(NW, nch, CHUNK)
    out_s, out_p = _embed_pair(sid, ph, speaker_id_table, phone_table)
    return (out_s.reshape(b, l, HIDDEN), out_p.reshape(b, l, HIDDEN))
